# baseline (speedup 1.0000x reference)
import functools

import jax
import jax.numpy as jnp
from jax import lax
from jax.experimental import pallas as pl
from jax.experimental.pallas import tpu as pltpu
from jax.experimental.pallas import tpu_sc as plsc

_B = 16384
_D = 64
_H = 128


_CH = 128


@functools.cache
def _gather_fn(B, D, NC, NS):
    NW = NC * NS
    b_per_w = B // NW
    n_ch = b_per_w // _CH
    mesh = plsc.VectorSubcoreMesh(core_axis_name="c", subcore_axis_name="s")

    @functools.partial(
        pl.kernel,
        out_type=(
            jax.ShapeDtypeStruct((B, D), jnp.float32),
            jax.ShapeDtypeStruct((B, D), jnp.float32),
        ),
        mesh=mesh,
        compiler_params=pltpu.CompilerParams(use_tc_tiling_on_sc=False),
        scratch_types=[
            pltpu.VMEM((n_ch, _CH), jnp.int32),
            pltpu.VMEM((n_ch, _CH), jnp.int32),
            pltpu.VMEM((b_per_w, D), jnp.float32),
            pltpu.VMEM((b_per_w, D), jnp.float32),
            pltpu.SemaphoreType.DMA,
            pltpu.SemaphoreType.DMA,
        ],
    )
    def gather_k(user_hbm, item_hbm, ut_hbm, it_hbm, ue_out, ie_out,
                 uidx_v, iidx_v, urows_v, irows_v, usem, isem):
        wid = lax.axis_index("s") * NC + lax.axis_index("c")
        base = wid * b_per_w
        pltpu.sync_copy(user_hbm.at[pl.ds(wid * n_ch, n_ch)], uidx_v)
        pltpu.sync_copy(item_hbm.at[pl.ds(wid * n_ch, n_ch)], iidx_v)
        copies = []
        for j in range(n_ch):
            copies.append(pltpu.async_copy(
                ut_hbm.at[uidx_v.at[j]], urows_v.at[pl.ds(j * _CH, _CH)], usem))
        for j in range(n_ch):
            copies.append(pltpu.async_copy(
                it_hbm.at[iidx_v.at[j]], irows_v.at[pl.ds(j * _CH, _CH)], isem))
        for c in copies:
            c.wait()
        pltpu.sync_copy(urows_v, ue_out.at[pl.ds(base, b_per_w)])
        pltpu.sync_copy(irows_v, ie_out.at[pl.ds(base, b_per_w)])

    return gather_k


def _mlp_body(ue_ref, ie_ref, w1u_ref, w1i_ref, b1_ref, w2_ref, b2_ref, out_ref):
    h = (jnp.dot(ue_ref[...], w1u_ref[...], preferred_element_type=jnp.float32)
         + jnp.dot(ie_ref[...], w1i_ref[...], preferred_element_type=jnp.float32)
         + b1_ref[...])
    h = jnp.maximum(h, 0.0)
    out_ref[...] = (
        jnp.dot(h, w2_ref[...], preferred_element_type=jnp.float32) + b2_ref[...]
    )


@functools.cache
def _mlp_fn(B, D, H, blk):
    grid = B // blk
    return pl.pallas_call(
        _mlp_body,
        grid=(grid,),
        in_specs=[
            pl.BlockSpec((blk, D), lambda i: (i, 0)),
            pl.BlockSpec((blk, D), lambda i: (i, 0)),
            pl.BlockSpec((D, H), lambda i: (0, 0)),
            pl.BlockSpec((D, H), lambda i: (0, 0)),
            pl.BlockSpec((1, H), lambda i: (0, 0)),
            pl.BlockSpec((H, 1), lambda i: (0, 0)),
            pl.BlockSpec((1, 1), lambda i: (0, 0)),
        ],
        out_specs=pl.BlockSpec((blk, 1), lambda i: (i, 0)),
        out_shape=jax.ShapeDtypeStruct((B, 1), jnp.float32),
    )


def kernel(user, item, user_table, item_table, W1, b1, W2, b2):
    info = plsc.get_sparse_core_info()
    ue, ie = _gather_fn(_B, _D, info.num_cores, info.num_subcores)(
        user.reshape(_B // _CH, _CH), item.reshape(_B // _CH, _CH),
        user_table, item_table)
    w1u = W1[:, :_D].T
    w1i = W1[:, _D:].T
    out = _mlp_fn(_B, _D, _H, 2048)(
        ue, ie, w1u, w1i, b1.reshape(1, _H), W2.T, b2.reshape(1, 1))
    return out.reshape(_B)
